# baseline (speedup 1.0000x reference)
import jax
import jax.numpy as jnp
from jax.experimental import pallas as pl
from jax.experimental.pallas import tpu as pltpu

_S = 3
_B = 8192
_D = 1024
_G = 64
_d = 16
_NS_ITERS = 8

_RB1 = 1024
_R1 = _B // _RB1


def _make_m(gram, cs, perm_row):
    f32 = jnp.float32
    bf16 = jnp.bfloat16

    mu = cs * (1.0 / _B)
    mh_bf = mu.astype(bf16)
    ml_bf = (mu - mh_bf.astype(f32)).astype(bf16)
    _outer = lambda a, b: jax.lax.dot_general(
        a, b, (((0,), (0,)), ((), ())), preferred_element_type=f32)
    mumu = (_outer(mh_bf, mh_bf) + _outer(mh_bf, ml_bf)
            + _outer(ml_bf, mh_bf))
    covf = gram * (1.0 / _B) - mumu

    iota_r = jax.lax.broadcasted_iota(jnp.int32, (_D, _D), 0)
    iota_c = jax.lax.broadcasted_iota(jnp.int32, (_D, _D), 1)
    p_row = jnp.broadcast_to(perm_row, (_D, _D))
    e_bf = jnp.where(iota_r == p_row, f32(1), f32(0)).astype(bf16)

    _NP = _D // 128
    _psl = [slice(q * 128, (q + 1) * 128) for q in range(_NP)]

    u = jax.lax.dot_general(covf.astype(bf16), e_bf, (((1,), (0,)), ((), ())),
                            preferred_element_type=f32)
    u_bf = u.astype(bf16)

    a2_parts = []
    for q in range(_NP):
        pq = jax.lax.dot_general(e_bf[:, _psl[q]], u_bf[:, _psl[q]],
                                 (((0,), (0,)), ((), ())),
                                 preferred_element_type=f32)
        a2_parts.append(jnp.concatenate(
            [pq[h * _d:(h + 1) * _d, h * _d:(h + 1) * _d]
             for h in range(128 // _d)], axis=1))
    a2 = jnp.concatenate(a2_parts, axis=1)

    pr = jax.lax.broadcasted_iota(jnp.int32, (128, 128), 0)
    pc = jax.lax.broadcasted_iota(jnp.int32, (128, 128), 1)
    s128_bf = jnp.where((pr >> 4) == (pc >> 4), f32(1), f32(0)).astype(bf16)

    rs = jnp.sum(a2 * a2, axis=0, keepdims=True)
    rs_bf = rs.astype(bf16)
    nrm = jnp.concatenate(
        [jax.lax.dot_general(rs_bf[:, s], s128_bf, (((1,), (0,)), ((), ())),
                             preferred_element_type=f32) for s in _psl],
        axis=1) * 1.02
    inv_nrm = 1.0 / nrm

    eye2 = (jax.lax.broadcasted_iota(jnp.int32, (_d, _D), 0) ==
            (jax.lax.broadcasted_iota(jnp.int32, (_d, _D), 1) & (_d - 1))
            ).astype(f32)

    def _bmm(p_bf, q_bf):
        outs = []
        for sl in _psl:
            bd = pltpu.repeat(q_bf[:, sl], 128 // _d, axis=0) * s128_bf
            outs.append(jax.lax.dot_general(
                p_bf[:, sl], bd, (((1,), (0,)), ((), ())),
                preferred_element_type=f32))
        return jnp.concatenate(outs, axis=1)

    y = a2 * inv_nrm
    z = eye2
    for _ in range(_NS_ITERS):
        y_bf = y.astype(bf16)
        t = 1.5 * eye2 - 0.5 * _bmm(z.astype(bf16), y_bf)
        t_bf = t.astype(bf16)
        y = _bmm(t_bf, y_bf)
        z = _bmm(t_bf, z.astype(bf16))
    w2 = z * jax.lax.rsqrt(nrm)

    w2_bf = w2.astype(bf16)
    ew = jnp.concatenate(
        [jax.lax.dot_general(
            e_bf[:, sl],
            pltpu.repeat(w2_bf[:, sl], 128 // _d, axis=0) * s128_bf,
            (((1,), (0,)), ((), ())), preferred_element_type=f32)
         for sl in _psl], axis=1)
    m = jax.lax.dot_general(ew.astype(bf16), e_bf, (((1,), (1,)), ((), ())),
                            preferred_element_type=f32)
    return m.astype(bf16)


def _fused_kernel(x_ref, perm_ref, y_ref, gram_ref, cs_ref, m_ref):
    r = pl.program_id(1)

    @pl.when(r < _R1)
    def _():
        xb = x_ref[...]
        xh = xb.astype(jnp.bfloat16)
        g = jax.lax.dot_general(
            xh, xh, (((0,), (0,)), ((), ())),
            preferred_element_type=jnp.float32)
        cs = jnp.sum(xb, axis=0, keepdims=True)

        @pl.when(r == 0)
        def _():
            gram_ref[...] = g
            cs_ref[...] = cs

        @pl.when(r != 0)
        def _():
            gram_ref[...] += g
            cs_ref[...] += cs

        @pl.when(r == _R1 - 1)
        def _():
            m_ref[...] = _make_m(gram_ref[...], cs_ref[...], perm_ref[0])

    @pl.when(r >= _R1)
    def _():
        xc = x_ref[...] - cs_ref[...] * (1.0 / _B)
        y_ref[...] = jax.lax.dot_general(
            xc.astype(jnp.bfloat16), m_ref[...],
            (((1,), (0,)), ((), ())),
            preferred_element_type=jnp.float32)


def kernel(x, perms):
    perms3 = perms.astype(jnp.int32)[:, None, :]
    return pl.pallas_call(
        _fused_kernel,
        grid=(_S, 2 * _R1),
        in_specs=[
            pl.BlockSpec(
                (_RB1, _D),
                lambda s, r: (s * _R1 + jnp.where(r < _R1, r, r - _R1), 0)),
            pl.BlockSpec((1, 1, _D), lambda s, r: (s, 0, 0)),
        ],
        out_specs=pl.BlockSpec(
            (_RB1, _D),
            lambda s, r: (s * _R1 + jnp.where(r < _R1, 0, r - _R1), 0)),
        out_shape=jax.ShapeDtypeStruct((_S * _B, _D), jnp.float32),
        scratch_shapes=[
            pltpu.VMEM((_D, _D), jnp.float32),
            pltpu.VMEM((1, _D), jnp.float32),
            pltpu.VMEM((_D, _D), jnp.bfloat16),
        ],
        compiler_params=pltpu.CompilerParams(
            dimension_semantics=("arbitrary", "arbitrary"),
            vmem_limit_bytes=56 * 1024 * 1024,
        ),
    )(x, perms3)


# speedup vs baseline: 1.0826x; 1.0826x over previous
import jax
import jax.numpy as jnp
from jax.experimental import pallas as pl
from jax.experimental.pallas import tpu as pltpu

_S = 3
_B = 8192
_D = 1024
_G = 64
_d = 16
_NS_ITERS = 8

_RB1 = 2048
_R1 = _B // _RB1


def _make_m(gram, cs, perm_row):
    f32 = jnp.float32
    bf16 = jnp.bfloat16

    mu = cs * (1.0 / _B)
    mh_bf = mu.astype(bf16)
    ml_bf = (mu - mh_bf.astype(f32)).astype(bf16)
    _outer = lambda a, b: jax.lax.dot_general(
        a, b, (((0,), (0,)), ((), ())), preferred_element_type=f32)
    mumu = (_outer(mh_bf, mh_bf) + _outer(mh_bf, ml_bf)
            + _outer(ml_bf, mh_bf))
    covf = gram * (1.0 / _B) - mumu

    iota_r = jax.lax.broadcasted_iota(jnp.int32, (_D, _D), 0)
    iota_c = jax.lax.broadcasted_iota(jnp.int32, (_D, _D), 1)
    p_row = jnp.broadcast_to(perm_row, (_D, _D))
    e_bf = jnp.where(iota_r == p_row, f32(1), f32(0)).astype(bf16)

    _NP = _D // 128
    _psl = [slice(q * 128, (q + 1) * 128) for q in range(_NP)]

    u = jax.lax.dot_general(covf.astype(bf16), e_bf, (((1,), (0,)), ((), ())),
                            preferred_element_type=f32)
    u_bf = u.astype(bf16)

    a2_parts = []
    for q in range(_NP):
        pq = jax.lax.dot_general(e_bf[:, _psl[q]], u_bf[:, _psl[q]],
                                 (((0,), (0,)), ((), ())),
                                 preferred_element_type=f32)
        a2_parts.append(jnp.concatenate(
            [pq[h * _d:(h + 1) * _d, h * _d:(h + 1) * _d]
             for h in range(128 // _d)], axis=1))
    a2 = jnp.concatenate(a2_parts, axis=1)

    pr = jax.lax.broadcasted_iota(jnp.int32, (128, 128), 0)
    pc = jax.lax.broadcasted_iota(jnp.int32, (128, 128), 1)
    s128_bf = jnp.where((pr >> 4) == (pc >> 4), f32(1), f32(0)).astype(bf16)

    rs = jnp.sum(a2 * a2, axis=0, keepdims=True)
    rs_bf = rs.astype(bf16)
    nrm = jnp.concatenate(
        [jax.lax.dot_general(rs_bf[:, s], s128_bf, (((1,), (0,)), ((), ())),
                             preferred_element_type=f32) for s in _psl],
        axis=1) * 1.02
    inv_nrm = 1.0 / nrm

    eye2 = (jax.lax.broadcasted_iota(jnp.int32, (_d, _D), 0) ==
            (jax.lax.broadcasted_iota(jnp.int32, (_d, _D), 1) & (_d - 1))
            ).astype(f32)

    def _bmm(p_bf, q_bf):
        outs = []
        for sl in _psl:
            bd = pltpu.repeat(q_bf[:, sl], 128 // _d, axis=0) * s128_bf
            outs.append(jax.lax.dot_general(
                p_bf[:, sl], bd, (((1,), (0,)), ((), ())),
                preferred_element_type=f32))
        return jnp.concatenate(outs, axis=1)

    y = a2 * inv_nrm
    z = eye2
    for _ in range(_NS_ITERS):
        y_bf = y.astype(bf16)
        t = 1.5 * eye2 - 0.5 * _bmm(z.astype(bf16), y_bf)
        t_bf = t.astype(bf16)
        y = _bmm(t_bf, y_bf)
        z = _bmm(t_bf, z.astype(bf16))
    w2 = z * jax.lax.rsqrt(nrm)

    w2_bf = w2.astype(bf16)
    ew = jnp.concatenate(
        [jax.lax.dot_general(
            e_bf[:, sl],
            pltpu.repeat(w2_bf[:, sl], 128 // _d, axis=0) * s128_bf,
            (((1,), (0,)), ((), ())), preferred_element_type=f32)
         for sl in _psl], axis=1)
    m = jax.lax.dot_general(ew.astype(bf16), e_bf, (((1,), (1,)), ((), ())),
                            preferred_element_type=f32)
    return m.astype(bf16)


def _fused_kernel(x_ref, perm_ref, y_ref, gram_ref, cs_ref, m_ref):
    r = pl.program_id(1)

    @pl.when(r < _R1)
    def _():
        xb = x_ref[...]
        xh = xb.astype(jnp.bfloat16)
        g = jax.lax.dot_general(
            xh, xh, (((0,), (0,)), ((), ())),
            preferred_element_type=jnp.float32)
        cs = jnp.sum(xb, axis=0, keepdims=True)

        @pl.when(r == 0)
        def _():
            gram_ref[...] = g
            cs_ref[...] = cs

        @pl.when(r != 0)
        def _():
            gram_ref[...] += g
            cs_ref[...] += cs

        @pl.when(r == _R1 - 1)
        def _():
            m_ref[...] = _make_m(gram_ref[...], cs_ref[...], perm_ref[0])

    @pl.when(r >= _R1)
    def _():
        xc = x_ref[...] - cs_ref[...] * (1.0 / _B)
        y_ref[...] = jax.lax.dot_general(
            xc.astype(jnp.bfloat16), m_ref[...],
            (((1,), (0,)), ((), ())),
            preferred_element_type=jnp.float32)


def kernel(x, perms):
    perms3 = perms.astype(jnp.int32)[:, None, :]
    return pl.pallas_call(
        _fused_kernel,
        grid=(_S, 2 * _R1),
        in_specs=[
            pl.BlockSpec(
                (_RB1, _D),
                lambda s, r: (s * _R1 + jnp.where(r < _R1, r, r - _R1), 0)),
            pl.BlockSpec((1, 1, _D), lambda s, r: (s, 0, 0)),
        ],
        out_specs=pl.BlockSpec(
            (_RB1, _D),
            lambda s, r: (s * _R1 + jnp.where(r < _R1, 0, r - _R1), 0)),
        out_shape=jax.ShapeDtypeStruct((_S * _B, _D), jnp.float32),
        scratch_shapes=[
            pltpu.VMEM((_D, _D), jnp.float32),
            pltpu.VMEM((1, _D), jnp.float32),
            pltpu.VMEM((_D, _D), jnp.bfloat16),
        ],
        compiler_params=pltpu.CompilerParams(
            dimension_semantics=("arbitrary", "arbitrary"),
            vmem_limit_bytes=56 * 1024 * 1024,
        ),
    )(x, perms3)
